# Initial kernel scaffold; baseline (speedup 1.0000x reference)
import functools

import jax
import jax.numpy as jnp
from jax import lax
from jax.experimental import pallas as pl
from jax.experimental.pallas import tpu as pltpu
from jax.experimental.pallas import tpu_sc as plsc

N = 50000
E = 800000
M = 16
H = 16
T = 16
C = 64

NC = 2
NS = 16
NW = NC * NS
L = 16
CH = 128
RPB = 25
IB = RPB * CH
EPW = 25600
NB = EPW // IB
E_PAD = NW * EPW
RPS = 3128
N_PAD = NS * RPS

BN = 1000


def _matmul_body(x_ref, w_ref, o_ref):
    o_ref[...] = jnp.dot(x_ref[...], w_ref[...],
                         preferred_element_type=jnp.float32)


def _edge_messages_table(features, w2):
    return pl.pallas_call(
        _matmul_body,
        grid=(N // BN,),
        in_specs=[
            pl.BlockSpec((BN, H), lambda i: (i, 0)),
            pl.BlockSpec((H, T * M), lambda i: (0, 0)),
        ],
        out_specs=pl.BlockSpec((BN, T * M), lambda i: (i, 0)),
        out_shape=jax.ShapeDtypeStruct((N, T * M), jnp.float32),
    )(features, w2)


@functools.partial(
    pl.kernel,
    out_type=jax.ShapeDtypeStruct((NC, N_PAD, M), jnp.float32),
    mesh=plsc.VectorSubcoreMesh(core_axis_name="c", subcore_axis_name="s",
                                num_cores=NC, num_subcores=NS),
    scratch_types=[
        pltpu.VMEM((IB,), jnp.int32),
        pltpu.VMEM((IB,), jnp.int32),
        pltpu.VMEM((RPB, CH), jnp.int32),
        pltpu.VMEM((RPB, CH), jnp.int32),
        pltpu.VMEM((CH, M), jnp.float32),
        pltpu.VMEM_SHARED((N_PAD, M), jnp.float32),
        pltpu.SemaphoreType.DMA,
    ],
)
def _sc_gather_scatter(g_hbm, src_hbm, typ_hbm, dst2_hbm, zero_hbm, out_hbm,
                       srcv, typv, gidx, dstv, rows, m_sh, sem):
    c = lax.axis_index("c")
    s = lax.axis_index("s")
    wid = s * NC + c

    pltpu.sync_copy(zero_hbm.at[pl.ds(s * RPS, RPS)],
                    m_sh.at[pl.ds(s * RPS, RPS)])
    plsc.subcore_barrier()

    def outer(b, carry):
        base = wid * EPW + b * IB
        pltpu.sync_copy(src_hbm.at[pl.ds(base, IB)], srcv)
        pltpu.sync_copy(typ_hbm.at[pl.ds(base, IB)], typv)
        pltpu.sync_copy(dst2_hbm.at[pl.ds(base // CH, RPB)], dstv)

        def idx_body(i, carry2):
            sv = srcv[pl.ds(i * L, L)]
            tv = typv[pl.ds(i * L, L)]
            gidx[i // (CH // L), pl.ds((i % (CH // L)) * L, L)] = sv * T + tv
            return carry2
        lax.fori_loop(0, IB // L, idx_body, 0)

        def gs_body(j, carry2):
            pltpu.async_copy(g_hbm.at[gidx.at[j]], rows, sem).wait()
            pltpu.sync_copy(rows, m_sh.at[dstv.at[j]], add=True)
            return carry2
        lax.fori_loop(0, RPB, gs_body, 0)
        return carry

    lax.fori_loop(0, NB, outer, 0)
    plsc.subcore_barrier()
    pltpu.sync_copy(m_sh.at[pl.ds(s * RPS, RPS)],
                    out_hbm.at[c, pl.ds(s * RPS, RPS)])


def _gru_body(p0_ref, p1_ref, f_ref, wi_ref, wh_ref, wo_ref,
              bi_ref, bh_ref, bo_ref, o_ref):
    m = p0_ref[...] + p1_ref[...]
    f = f_ref[...]
    gi = jnp.dot(m, wi_ref[...], preferred_element_type=jnp.float32) + bi_ref[...]
    gh = jnp.dot(f, wh_ref[...], preferred_element_type=jnp.float32) + bh_ref[...]
    r = jax.nn.sigmoid(gi[:, :H] + gh[:, :H])
    z = jax.nn.sigmoid(gi[:, H:2 * H] + gh[:, H:2 * H])
    n = jnp.tanh(gi[:, 2 * H:] + r * gh[:, 2 * H:])
    h_new = (1.0 - z) * n + z * f
    o_ref[...] = jnp.dot(h_new, wo_ref[...],
                         preferred_element_type=jnp.float32) + bo_ref[...]


def _gru_readout(p0, p1, features, wi_t, wh_t, wo_t, bi, bh, bo):
    full = lambda i: (0, 0)
    return pl.pallas_call(
        _gru_body,
        grid=(N // BN,),
        in_specs=[
            pl.BlockSpec((BN, M), lambda i: (i, 0)),
            pl.BlockSpec((BN, M), lambda i: (i, 0)),
            pl.BlockSpec((BN, H), lambda i: (i, 0)),
            pl.BlockSpec((M, 3 * H), full),
            pl.BlockSpec((H, 3 * H), full),
            pl.BlockSpec((H, C), full),
            pl.BlockSpec((1, 3 * H), full),
            pl.BlockSpec((1, 3 * H), full),
            pl.BlockSpec((1, C), full),
        ],
        out_specs=pl.BlockSpec((BN, C), lambda i: (i, 0)),
        out_shape=jax.ShapeDtypeStruct((N, C), jnp.float32),
    )(p0, p1, features, wi_t, wh_t, wo_t, bi, bh, bo)


def kernel(features, edge_index, edge_type, edge_table, W_ih, W_hh, b_ih,
           b_hh, W_out, b_out):
    src = edge_index[0]
    dst = edge_index[1]

    w2 = edge_table.reshape(T, M, H).transpose(2, 0, 1).reshape(H, T * M)
    wi_t = W_ih.T
    wh_t = W_hh.T
    wo_t = W_out.T
    bi = b_ih.reshape(1, 3 * H)
    bh = b_hh.reshape(1, 3 * H)
    bo = b_out.reshape(1, C)

    pad = E_PAD - E
    src1 = jnp.concatenate([src, jnp.zeros((pad,), jnp.int32)])
    typ1 = jnp.concatenate([edge_type, jnp.zeros((pad,), jnp.int32)])
    dst2 = jnp.concatenate([dst, jnp.full((pad,), N, jnp.int32)]
                           ).reshape(E_PAD // CH, CH)
    zero = jnp.zeros((N_PAD, M), jnp.float32)

    g2 = _edge_messages_table(features, w2)
    g1 = g2.reshape(N * T, M)

    partials = _sc_gather_scatter(g1, src1, typ1, dst2, zero)
    p0 = partials[0, :N]
    p1 = partials[1, :N]

    return _gru_readout(p0, p1, features, wi_t, wh_t, wo_t, bi, bh, bo)


# speedup vs baseline: 14.4464x; 14.4464x over previous
import functools

import jax
import jax.numpy as jnp
from jax import lax
from jax.experimental import pallas as pl
from jax.experimental.pallas import tpu as pltpu
from jax.experimental.pallas import tpu_sc as plsc

N = 50000
E = 800000
M = 16
H = 16
T = 16
C = 64

NC = 2
NS = 16
NW = NC * NS
L = 16
CH = 128
RPB = 25
IB = RPB * CH
EPW = 25600
NB = EPW // IB
E_PAD = NW * EPW
RPS = 3128
N_PAD = NS * RPS

BN = 1000


def _matmul_body(x_ref, w_ref, o_ref):
    o_ref[...] = jnp.dot(x_ref[...], w_ref[...],
                         preferred_element_type=jnp.float32)


def _edge_messages_table(features, w2):
    return pl.pallas_call(
        _matmul_body,
        grid=(N // BN,),
        in_specs=[
            pl.BlockSpec((BN, H), lambda i: (i, 0)),
            pl.BlockSpec((H, T * M), lambda i: (0, 0)),
        ],
        out_specs=pl.BlockSpec((BN, T * M), lambda i: (i, 0)),
        out_shape=jax.ShapeDtypeStruct((N, T * M), jnp.float32),
    )(features, w2)


def _sc_body(g_hbm, src_hbm, typ_hbm, dst_hbm, zero_hbm, out_hbm,
             srcv, typv, dst1v, gidx, dstv, rows, m_sh, sem):
    c = lax.axis_index("c")
    s = lax.axis_index("s")
    wid = s * NC + c

    pltpu.sync_copy(zero_hbm.at[pl.ds(s * RPS, RPS)],
                    m_sh.at[pl.ds(s * RPS, RPS)])
    plsc.subcore_barrier()

    def outer(b, carry):
        base = wid * EPW + b * IB
        pltpu.sync_copy(src_hbm.at[pl.ds(base, IB)], srcv)
        pltpu.sync_copy(typ_hbm.at[pl.ds(base, IB)], typv)
        pltpu.sync_copy(dst_hbm.at[pl.ds(base, IB)], dst1v)

        def idx_body(i, carry2):
            sv = srcv[pl.ds(i * L, L)]
            tv = typv[pl.ds(i * L, L)]
            dv = dst1v[pl.ds(i * L, L)]
            gidx[i // (CH // L), pl.ds((i % (CH // L)) * L, L)] = sv * T + tv
            dstv[i // (CH // L), pl.ds((i % (CH // L)) * L, L)] = dv
            return carry2
        lax.fori_loop(0, IB // L, idx_body, 0)

        def gs_body(j, carry2):
            pltpu.async_copy(g_hbm.at[gidx.at[j]], rows, sem).wait()
            pltpu.sync_copy(rows, m_sh.at[dstv.at[j]], add=True)
            return carry2
        lax.fori_loop(0, RPB, gs_body, 0)
        return carry

    lax.fori_loop(0, NB, outer, 0)
    plsc.subcore_barrier()
    pltpu.sync_copy(m_sh.at[pl.ds(s * RPS, RPS)],
                    out_hbm.at[c, pl.ds(s * RPS, RPS)])


@functools.lru_cache(maxsize=1)
def _sc_gather_scatter():
    return pl.kernel(
        _sc_body,
        out_type=jax.ShapeDtypeStruct((NC, N_PAD, M), jnp.float32),
        mesh=plsc.VectorSubcoreMesh(core_axis_name="c", subcore_axis_name="s",
                                    num_cores=NC, num_subcores=NS),
        scratch_types=[
            pltpu.VMEM((IB,), jnp.int32),
            pltpu.VMEM((IB,), jnp.int32),
            pltpu.VMEM((IB,), jnp.int32),
            pltpu.VMEM((RPB, CH), jnp.int32),
            pltpu.VMEM((RPB, CH), jnp.int32),
            pltpu.VMEM((CH, M), jnp.float32),
            pltpu.VMEM_SHARED((N_PAD, M), jnp.float32),
            pltpu.SemaphoreType.DMA,
        ],
        compiler_params=pltpu.CompilerParams(use_tc_tiling_on_sc=False),
    )


def _gru_body(p0_ref, p1_ref, f_ref, wi_ref, wh_ref, wo_ref,
              bi_ref, bh_ref, bo_ref, o_ref):
    m = p0_ref[...] + p1_ref[...]
    f = f_ref[...]
    gi = jnp.dot(m, wi_ref[...], preferred_element_type=jnp.float32) + bi_ref[...]
    gh = jnp.dot(f, wh_ref[...], preferred_element_type=jnp.float32) + bh_ref[...]
    r = jax.nn.sigmoid(gi[:, :H] + gh[:, :H])
    z = jax.nn.sigmoid(gi[:, H:2 * H] + gh[:, H:2 * H])
    n = jnp.tanh(gi[:, 2 * H:] + r * gh[:, 2 * H:])
    h_new = (1.0 - z) * n + z * f
    o_ref[...] = jnp.dot(h_new, wo_ref[...],
                         preferred_element_type=jnp.float32) + bo_ref[...]


def _gru_readout(p0, p1, features, wi_t, wh_t, wo_t, bi, bh, bo):
    full = lambda i: (0, 0)
    return pl.pallas_call(
        _gru_body,
        grid=(N // BN,),
        in_specs=[
            pl.BlockSpec((BN, M), lambda i: (i, 0)),
            pl.BlockSpec((BN, M), lambda i: (i, 0)),
            pl.BlockSpec((BN, H), lambda i: (i, 0)),
            pl.BlockSpec((M, 3 * H), full),
            pl.BlockSpec((H, 3 * H), full),
            pl.BlockSpec((H, C), full),
            pl.BlockSpec((1, 3 * H), full),
            pl.BlockSpec((1, 3 * H), full),
            pl.BlockSpec((1, C), full),
        ],
        out_specs=pl.BlockSpec((BN, C), lambda i: (i, 0)),
        out_shape=jax.ShapeDtypeStruct((N, C), jnp.float32),
    )(p0, p1, features, wi_t, wh_t, wo_t, bi, bh, bo)


def kernel(features, edge_index, edge_type, edge_table, W_ih, W_hh, b_ih,
           b_hh, W_out, b_out):
    src = edge_index[0]
    dst = edge_index[1]

    w2 = edge_table.reshape(T, M, H).transpose(2, 0, 1).reshape(H, T * M)
    wi_t = W_ih.T
    wh_t = W_hh.T
    wo_t = W_out.T
    bi = b_ih.reshape(1, 3 * H)
    bh = b_hh.reshape(1, 3 * H)
    bo = b_out.reshape(1, C)

    pad = E_PAD - E
    src1 = jnp.concatenate([src, jnp.zeros((pad,), jnp.int32)])
    typ1 = jnp.concatenate([edge_type, jnp.zeros((pad,), jnp.int32)])
    dst1 = jnp.concatenate([dst, jnp.full((pad,), N, jnp.int32)])
    zero = jnp.zeros((N_PAD, M), jnp.float32)

    g2 = _edge_messages_table(features, w2)
    g1 = g2.reshape(N * T, M)

    partials = _sc_gather_scatter()(g1, src1, typ1, dst1, zero)
    p0 = partials[0, :N]
    p1 = partials[1, :N]

    return _gru_readout(p0, p1, features, wi_t, wh_t, wo_t, bi, bh, bo)


# speedup vs baseline: 20.7768x; 1.4382x over previous
import functools

import jax
import jax.numpy as jnp
from jax import lax
from jax.experimental import pallas as pl
from jax.experimental.pallas import tpu as pltpu
from jax.experimental.pallas import tpu_sc as plsc

N = 50000
E = 800000
M = 16
H = 16
T = 16
C = 64

NC = 2
NS = 16
NW = NC * NS
L = 16
CH = 128
RPB = 25
IB = RPB * CH
EPW = 25600
NB = EPW // IB
E_PAD = NW * EPW
RPS = 3128
N_PAD = NS * RPS

PK = 8
BP = 128
NPK = N // PK
GB = (NPK + BP - 1) // BP


def _msg_table_body(f_ref, w_ref, o_ref):
    f = f_ref[...]
    for k in range(PK):
        o_ref[0, :, k, :] = jnp.dot(f, w_ref[0, :, k * 128:(k + 1) * 128],
                                    preferred_element_type=jnp.float32)


def _msg_table(fp, w2blk):
    return pl.pallas_call(
        _msg_table_body,
        grid=(2, GB),
        in_specs=[
            pl.BlockSpec((BP, 128), lambda h, j: (j, 0)),
            pl.BlockSpec((1, 128, PK * 128), lambda h, j: (h, 0, 0)),
        ],
        out_specs=pl.BlockSpec((1, BP, PK, 128), lambda h, j: (h, j, 0, 0)),
        out_shape=jax.ShapeDtypeStruct((2, NPK, PK, 128), jnp.float32),
    )(fp, w2blk)


def _sc_body(g_hbm, src_hbm, typ_hbm, dst_hbm, zero_hbm, out_hbm,
             srcv, typv, dst1v, gidx, dstv, rows0, rows1, m_sh, sem0, sem1):
    c = lax.axis_index("c")
    s = lax.axis_index("s")
    wid = s * NC + c

    pltpu.sync_copy(zero_hbm.at[pl.ds(s * RPS, RPS)],
                    m_sh.at[pl.ds(s * RPS, RPS)])
    plsc.subcore_barrier()

    def outer(b, carry):
        base = wid * EPW + b * IB
        pltpu.sync_copy(src_hbm.at[pl.ds(base, IB)], srcv)
        pltpu.sync_copy(typ_hbm.at[pl.ds(base, IB)], typv)
        pltpu.sync_copy(dst_hbm.at[pl.ds(base, IB)], dst1v)

        def idx_body(i, carry2):
            sv = srcv[pl.ds(i * L, L)]
            tv = typv[pl.ds(i * L, L)]
            dv = dst1v[pl.ds(i * L, L)]
            g = sv * PK + (tv & (PK - 1)) + (tv >> 3) * (N * PK)
            gidx[i // (CH // L), pl.ds((i % (CH // L)) * L, L)] = g
            dstv[i // (CH // L), pl.ds((i % (CH // L)) * L, L)] = dv
            return carry2
        lax.fori_loop(0, IB // L, idx_body, 0)

        pltpu.async_copy(g_hbm.at[gidx.at[0]], rows0, sem0)

        def gs_pair(p, carry2):
            j0 = 2 * p
            pltpu.async_copy(g_hbm.at[gidx.at[j0 + 1]], rows1, sem1)
            pltpu.make_async_copy(g_hbm.at[gidx.at[j0]], rows0, sem0).wait()
            pltpu.sync_copy(rows0, m_sh.at[dstv.at[j0]], add=True)
            pltpu.async_copy(g_hbm.at[gidx.at[j0 + 2]], rows0, sem0)
            pltpu.make_async_copy(g_hbm.at[gidx.at[j0 + 1]], rows1, sem1).wait()
            pltpu.sync_copy(rows1, m_sh.at[dstv.at[j0 + 1]], add=True)
            return carry2
        lax.fori_loop(0, (RPB - 1) // 2, gs_pair, 0)
        pltpu.make_async_copy(g_hbm.at[gidx.at[RPB - 1]], rows0, sem0).wait()
        pltpu.sync_copy(rows0, m_sh.at[dstv.at[RPB - 1]], add=True)
        return carry

    lax.fori_loop(0, NB, outer, 0)
    plsc.subcore_barrier()
    pltpu.sync_copy(m_sh.at[pl.ds(s * RPS, RPS)],
                    out_hbm.at[c, pl.ds(s * RPS, RPS)])


@functools.lru_cache(maxsize=1)
def _sc_gather_scatter():
    return pl.kernel(
        _sc_body,
        out_type=jax.ShapeDtypeStruct((NC, N_PAD, M), jnp.float32),
        mesh=plsc.VectorSubcoreMesh(core_axis_name="c", subcore_axis_name="s",
                                    num_cores=NC, num_subcores=NS),
        scratch_types=[
            pltpu.VMEM((IB,), jnp.int32),
            pltpu.VMEM((IB,), jnp.int32),
            pltpu.VMEM((IB,), jnp.int32),
            pltpu.VMEM((RPB, CH), jnp.int32),
            pltpu.VMEM((RPB, CH), jnp.int32),
            pltpu.VMEM((CH, M), jnp.float32),
            pltpu.VMEM((CH, M), jnp.float32),
            pltpu.VMEM_SHARED((N_PAD, M), jnp.float32),
            pltpu.SemaphoreType.DMA,
            pltpu.SemaphoreType.DMA,
        ],
        compiler_params=pltpu.CompilerParams(use_tc_tiling_on_sc=False),
    )


def _gru_body(p_ref, f_ref, wi_ref, wh_ref, wo_ref, bi_ref, bh_ref, bo_ref,
              o_ref):
    m = p_ref[0] + p_ref[1]
    f = f_ref[...]
    gi = jnp.dot(m, wi_ref[...], preferred_element_type=jnp.float32) + bi_ref[...]
    gh = jnp.dot(f, wh_ref[...], preferred_element_type=jnp.float32) + bh_ref[...]
    r = jax.nn.sigmoid(gi[:, :128] + gh[:, :128])
    z = jax.nn.sigmoid(gi[:, 128:256] + gh[:, 128:256])
    n = jnp.tanh(gi[:, 256:] + r * gh[:, 256:])
    h_new = (1.0 - z) * n + z * f
    o_ref[...] = jnp.dot(h_new, wo_ref[...],
                         preferred_element_type=jnp.float32) + bo_ref[...]


def _gru_readout(p2, fp, wi_blk, wh_blk, wo_blk, bi_p, bh_p, bo_p):
    full = lambda j: (0, 0)
    return pl.pallas_call(
        _gru_body,
        grid=(GB,),
        in_specs=[
            pl.BlockSpec((2, BP, 128), lambda j: (0, j, 0)),
            pl.BlockSpec((BP, 128), lambda j: (j, 0)),
            pl.BlockSpec((128, 3 * 128), full),
            pl.BlockSpec((128, 3 * 128), full),
            pl.BlockSpec((128, PK * C), full),
            pl.BlockSpec((1, 3 * 128), full),
            pl.BlockSpec((1, 3 * 128), full),
            pl.BlockSpec((1, PK * C), full),
        ],
        out_specs=pl.BlockSpec((BP, PK * C), lambda j: (j, 0)),
        out_shape=jax.ShapeDtypeStruct((NPK, PK * C), jnp.float32),
    )(p2, fp, wi_blk, wh_blk, wo_blk, bi_p, bh_p, bo_p)


def kernel(features, edge_index, edge_type, edge_table, W_ih, W_hh, b_ih,
           b_hh, W_out, b_out):
    src = edge_index[0]
    dst = edge_index[1]

    eye8 = jnp.eye(PK, dtype=jnp.float32)
    w2 = edge_table.reshape(T, M, H).transpose(2, 0, 1).reshape(H, T * M)
    w2blk = jnp.stack([jnp.kron(eye8, w2[:, :128]),
                       jnp.kron(eye8, w2[:, 128:])])
    wi_t = W_ih.T
    wh_t = W_hh.T
    wi_blk = jnp.concatenate(
        [jnp.kron(eye8, wi_t[:, g * H:(g + 1) * H]) for g in range(3)], axis=1)
    wh_blk = jnp.concatenate(
        [jnp.kron(eye8, wh_t[:, g * H:(g + 1) * H]) for g in range(3)], axis=1)
    wo_blk = jnp.kron(eye8, W_out.T)
    bi_p = jnp.tile(b_ih.reshape(3, 1, H), (1, PK, 1)).reshape(1, 3 * 128)
    bh_p = jnp.tile(b_hh.reshape(3, 1, H), (1, PK, 1)).reshape(1, 3 * 128)
    bo_p = jnp.tile(b_out.reshape(1, C), (PK, 1)).reshape(1, PK * C)

    pad = E_PAD - E
    src1 = jnp.concatenate([src, jnp.zeros((pad,), jnp.int32)])
    typ1 = jnp.concatenate([edge_type, jnp.zeros((pad,), jnp.int32)])
    dst1 = jnp.concatenate([dst, jnp.full((pad,), N, jnp.int32)])
    zero = jnp.zeros((N_PAD, M), jnp.float32)

    fp = features.reshape(NPK, 128)
    g = _msg_table(fp, w2blk)
    g1 = g.reshape(N * T, M)

    partials = _sc_gather_scatter()(g1, src1, typ1, dst1, zero)
    p2 = partials.reshape(NC, N_PAD * M // 128, 128)

    out_p = _gru_readout(p2, fp, wi_blk, wh_blk, wo_blk, bi_p, bh_p, bo_p)
    return out_p.reshape(N, C)


# speedup vs baseline: 23.0513x; 1.1095x over previous
import functools

import jax
import jax.numpy as jnp
from jax import lax
from jax.experimental import pallas as pl
from jax.experimental.pallas import tpu as pltpu
from jax.experimental.pallas import tpu_sc as plsc

N = 50000
E = 800000
M = 16
H = 16
T = 16
C = 64

NC = 2
NS = 16
NW = NC * NS
L = 16
CH = 128
RPB = 25
IB = RPB * CH
NB0 = 9
NB1 = 7
EPW0 = NB0 * IB
EPW1 = NB1 * IB
E_PAD = NS * (EPW0 + EPW1)
RPS = 3128
N_PAD = NS * RPS

PK = 8
BP = 128
NPK = N // PK
GB = (NPK + BP - 1) // BP


BN = 1000


def _msg_table_body(f_ref, w_ref, o_ref):
    f = f_ref[...]
    o_ref[0] = jnp.dot(f, w_ref[0], preferred_element_type=jnp.float32)
    o_ref[1] = jnp.dot(f, w_ref[1], preferred_element_type=jnp.float32)


def _msg_table(features, w2h):
    return pl.pallas_call(
        _msg_table_body,
        grid=(N // BN,),
        in_specs=[
            pl.BlockSpec((BN, H), lambda j: (j, 0)),
            pl.BlockSpec((2, H, 128), lambda j: (0, 0, 0)),
        ],
        out_specs=pl.BlockSpec((2, BN, 128), lambda j: (0, j, 0)),
        out_shape=jax.ShapeDtypeStruct((2, N, 128), jnp.float32),
    )(features, w2h)


def _sc_body(g_hbm, src_hbm, typ_hbm, dst_hbm, zero_hbm, out_hbm,
             srcv, typv, dst1v, gidx, dstv, rows0, rows1, m_sh, sem0, sem1):
    c = lax.axis_index("c")
    s = lax.axis_index("s")
    wbase = lax.select(c == 0, s * EPW0, NS * EPW0 + s * EPW1)
    nb = lax.select(c == 0, NB0, NB1)

    pltpu.sync_copy(zero_hbm.at[pl.ds(s * RPS, RPS)],
                    m_sh.at[pl.ds(s * RPS, RPS)])
    plsc.subcore_barrier()

    def outer(b, carry):
        base = wbase + b * IB
        pltpu.sync_copy(src_hbm.at[pl.ds(base, IB)], srcv)
        pltpu.sync_copy(typ_hbm.at[pl.ds(base, IB)], typv)
        pltpu.sync_copy(dst_hbm.at[pl.ds(base, IB)], dst1v)

        def idx_body(i, carry2):
            sv = srcv[pl.ds(i * L, L)]
            tv = typv[pl.ds(i * L, L)]
            dv = dst1v[pl.ds(i * L, L)]
            g = sv * PK + (tv & (PK - 1)) + (tv >> 3) * (N * PK)
            gidx[i // (CH // L), pl.ds((i % (CH // L)) * L, L)] = g
            dstv[i // (CH // L), pl.ds((i % (CH // L)) * L, L)] = dv
            return carry2
        lax.fori_loop(0, IB // L, idx_body, 0)

        pltpu.async_copy(g_hbm.at[gidx.at[0]], rows0, sem0)

        def gs_pair(p, carry2):
            j0 = 2 * p
            pltpu.async_copy(g_hbm.at[gidx.at[j0 + 1]], rows1, sem1)
            pltpu.make_async_copy(g_hbm.at[gidx.at[j0]], rows0, sem0).wait()
            pltpu.sync_copy(rows0, m_sh.at[dstv.at[j0]], add=True)
            pltpu.async_copy(g_hbm.at[gidx.at[j0 + 2]], rows0, sem0)
            pltpu.make_async_copy(g_hbm.at[gidx.at[j0 + 1]], rows1, sem1).wait()
            pltpu.sync_copy(rows1, m_sh.at[dstv.at[j0 + 1]], add=True)
            return carry2
        lax.fori_loop(0, (RPB - 1) // 2, gs_pair, 0)
        pltpu.make_async_copy(g_hbm.at[gidx.at[RPB - 1]], rows0, sem0).wait()
        pltpu.sync_copy(rows0, m_sh.at[dstv.at[RPB - 1]], add=True)
        return carry

    lax.fori_loop(0, nb, outer, 0)
    plsc.subcore_barrier()
    pltpu.sync_copy(m_sh.at[pl.ds(s * RPS, RPS)],
                    out_hbm.at[c, pl.ds(s * RPS, RPS)])


@functools.lru_cache(maxsize=1)
def _sc_gather_scatter():
    return pl.kernel(
        _sc_body,
        out_type=jax.ShapeDtypeStruct((NC, N_PAD, M), jnp.float32),
        mesh=plsc.VectorSubcoreMesh(core_axis_name="c", subcore_axis_name="s",
                                    num_cores=NC, num_subcores=NS),
        scratch_types=[
            pltpu.VMEM((IB,), jnp.int32),
            pltpu.VMEM((IB,), jnp.int32),
            pltpu.VMEM((IB,), jnp.int32),
            pltpu.VMEM((RPB, CH), jnp.int32),
            pltpu.VMEM((RPB, CH), jnp.int32),
            pltpu.VMEM((CH, M), jnp.float32),
            pltpu.VMEM((CH, M), jnp.float32),
            pltpu.VMEM_SHARED((N_PAD, M), jnp.float32),
            pltpu.SemaphoreType.DMA,
            pltpu.SemaphoreType.DMA,
        ],
        compiler_params=pltpu.CompilerParams(use_tc_tiling_on_sc=False),
    )


def _gru_body(p_ref, f_ref, wi_ref, wh_ref, wo_ref, bi_ref, bh_ref, bo_ref,
              o_ref):
    m = p_ref[0] + p_ref[1]
    f = f_ref[...]
    gi = jnp.dot(m, wi_ref[...], preferred_element_type=jnp.float32) + bi_ref[...]
    gh = jnp.dot(f, wh_ref[...], preferred_element_type=jnp.float32) + bh_ref[...]
    r = jax.nn.sigmoid(gi[:, :128] + gh[:, :128])
    z = jax.nn.sigmoid(gi[:, 128:256] + gh[:, 128:256])
    n = jnp.tanh(gi[:, 256:] + r * gh[:, 256:])
    h_new = (1.0 - z) * n + z * f
    o_ref[...] = jnp.dot(h_new, wo_ref[...],
                         preferred_element_type=jnp.float32) + bo_ref[...]


def _gru_readout(p2, fp, wi_blk, wh_blk, wo_blk, bi_p, bh_p, bo_p):
    full = lambda j: (0, 0)
    return pl.pallas_call(
        _gru_body,
        grid=(GB,),
        in_specs=[
            pl.BlockSpec((2, BP, 128), lambda j: (0, j, 0)),
            pl.BlockSpec((BP, 128), lambda j: (j, 0)),
            pl.BlockSpec((128, 3 * 128), full),
            pl.BlockSpec((128, 3 * 128), full),
            pl.BlockSpec((128, PK * C), full),
            pl.BlockSpec((1, 3 * 128), full),
            pl.BlockSpec((1, 3 * 128), full),
            pl.BlockSpec((1, PK * C), full),
        ],
        out_specs=pl.BlockSpec((BP, PK * C), lambda j: (j, 0)),
        out_shape=jax.ShapeDtypeStruct((NPK, PK * C), jnp.float32),
    )(p2, fp, wi_blk, wh_blk, wo_blk, bi_p, bh_p, bo_p)


def kernel(features, edge_index, edge_type, edge_table, W_ih, W_hh, b_ih,
           b_hh, W_out, b_out):
    src = edge_index[0]
    dst = edge_index[1]

    eye8 = jnp.eye(PK, dtype=jnp.float32)
    w2 = edge_table.reshape(T, M, H).transpose(2, 0, 1).reshape(H, T * M)
    w2h = jnp.stack([w2[:, :128], w2[:, 128:]])
    wi_t = W_ih.T
    wh_t = W_hh.T
    wi_blk = jnp.concatenate(
        [jnp.kron(eye8, wi_t[:, g * H:(g + 1) * H]) for g in range(3)], axis=1)
    wh_blk = jnp.concatenate(
        [jnp.kron(eye8, wh_t[:, g * H:(g + 1) * H]) for g in range(3)], axis=1)
    wo_blk = jnp.kron(eye8, W_out.T)
    bi_p = jnp.tile(b_ih.reshape(3, 1, H), (1, PK, 1)).reshape(1, 3 * 128)
    bh_p = jnp.tile(b_hh.reshape(3, 1, H), (1, PK, 1)).reshape(1, 3 * 128)
    bo_p = jnp.tile(b_out.reshape(1, C), (PK, 1)).reshape(1, PK * C)

    pad = E_PAD - E
    src1 = jnp.concatenate([src, jnp.zeros((pad,), jnp.int32)])
    typ1 = jnp.concatenate([edge_type, jnp.zeros((pad,), jnp.int32)])
    dst1 = jnp.concatenate([dst, jnp.full((pad,), N, jnp.int32)])
    zero = jnp.zeros((N_PAD, M), jnp.float32)

    fp = features.reshape(NPK, 128)
    g = _msg_table(features, w2h)
    g1 = g.reshape(N * T, M)

    partials = _sc_gather_scatter()(g1, src1, typ1, dst1, zero)
    p2 = partials.reshape(NC, N_PAD * M // 128, 128)

    out_p = _gru_readout(p2, fp, wi_blk, wh_blk, wo_blk, bi_p, bh_p, bo_p)
    return out_p.reshape(N, C)


# speedup vs baseline: 31.8748x; 1.3828x over previous
import functools

import jax
import jax.numpy as jnp
from jax import lax
from jax.experimental import pallas as pl
from jax.experimental.pallas import tpu as pltpu
from jax.experimental.pallas import tpu_sc as plsc

N = 50000
E = 800000
M = 16
H = 16
T = 16
C = 64

NC = 2
NS = 16
NW = NC * NS
L = 16
CH = 128
RPB = 28
IB = RPB * CH
PD = 4
NB0 = 8
NB1 = 6
EPW0 = NB0 * IB
EPW1 = NB1 * IB
E_PAD = NS * (EPW0 + EPW1)
RPS = 3128
N_PAD = NS * RPS

PK = 8
BP = 128
NPK = N // PK
GB = (NPK + BP - 1) // BP


BN = 1000


def _msg_table_body(f_ref, w_ref, o_ref):
    f = f_ref[...]
    o_ref[0] = jnp.dot(f, w_ref[0], preferred_element_type=jnp.float32)
    o_ref[1] = jnp.dot(f, w_ref[1], preferred_element_type=jnp.float32)


def _msg_table(features, w2h):
    return pl.pallas_call(
        _msg_table_body,
        grid=(N // BN,),
        in_specs=[
            pl.BlockSpec((BN, H), lambda j: (j, 0)),
            pl.BlockSpec((2, H, 128), lambda j: (0, 0, 0)),
        ],
        out_specs=pl.BlockSpec((2, BN, 128), lambda j: (0, j, 0)),
        out_shape=jax.ShapeDtypeStruct((2, N, 128), jnp.float32),
    )(features, w2h)


def _sc_body(g_hbm, ei_hbm, typ_hbm, zero_hbm, out_hbm,
             srcv, typv, dst1v, gidx, dstv, bufs, m_sh, gsems, ssems):
    c = lax.axis_index("c")
    s = lax.axis_index("s")
    wbase = lax.select(c == 0, s * EPW0, NS * EPW0 + s * EPW1)
    nb = lax.select(c == 0, NB0, NB1)

    pltpu.sync_copy(zero_hbm.at[pl.ds(s * RPS, RPS)],
                    m_sh.at[pl.ds(s * RPS, RPS)])
    plsc.subcore_barrier()

    def outer(b, carry):
        base = wbase + b * IB
        pltpu.sync_copy(ei_hbm.at[0, pl.ds(base, IB)], srcv)
        pltpu.sync_copy(ei_hbm.at[1, pl.ds(base, IB)], dst1v)
        pltpu.sync_copy(typ_hbm.at[pl.ds(base, IB)], typv)

        def idx_body(i, carry2):
            sv = srcv[pl.ds(i * L, L)]
            tv = typv[pl.ds(i * L, L)]
            dv = dst1v[pl.ds(i * L, L)]
            g = sv * PK + (tv & (PK - 1)) + (tv >> 3) * (N * PK)
            gidx[i // (CH // L), pl.ds((i % (CH // L)) * L, L)] = g
            dstv[i // (CH // L), pl.ds((i % (CH // L)) * L, L)] = dv
            return carry2
        lax.fori_loop(0, IB // L, idx_body, 0)

        for k in range(PD):
            pltpu.async_copy(g_hbm.at[gidx.at[k]], bufs[k], gsems[k])

        def group(q, carry2):
            j0 = q * PD
            for k in range(PD):
                pltpu.make_async_copy(g_hbm.at[gidx.at[j0 + k]], bufs[k],
                                      gsems[k]).wait()
                pltpu.async_copy(bufs[k], m_sh.at[dstv.at[j0 + k]], ssems[k],
                                 add=True)

            @pl.when(q < RPB // PD - 1)
            def _():
                for k in range(PD):
                    pltpu.make_async_copy(bufs[k], m_sh.at[dstv.at[j0 + k]],
                                          ssems[k]).wait()
                    pltpu.async_copy(g_hbm.at[gidx.at[j0 + PD + k]], bufs[k],
                                     gsems[k])
            return carry2
        lax.fori_loop(0, RPB // PD, group, 0)
        for k in range(PD):
            pltpu.make_async_copy(bufs[k], m_sh.at[dstv.at[RPB - PD + k]],
                                  ssems[k]).wait()
        return carry

    lax.fori_loop(0, nb, outer, 0)
    plsc.subcore_barrier()
    pltpu.sync_copy(m_sh.at[pl.ds(s * RPS, RPS)],
                    out_hbm.at[c, pl.ds(s * RPS, RPS)])


@functools.lru_cache(maxsize=1)
def _sc_gather_scatter():
    def body(g_hbm, ei_hbm, typ_hbm, zero_hbm, out_hbm, srcv, typv, dst1v,
             gidx, dstv, b0, b1, b2, b3, m_sh, g0, g1, g2, g3, s0, s1, s2, s3):
        _sc_body(g_hbm, ei_hbm, typ_hbm, zero_hbm, out_hbm, srcv, typv,
                 dst1v, gidx, dstv, [b0, b1, b2, b3], m_sh,
                 [g0, g1, g2, g3], [s0, s1, s2, s3])

    return pl.kernel(
        body,
        out_type=jax.ShapeDtypeStruct((NC, N_PAD, M), jnp.float32),
        mesh=plsc.VectorSubcoreMesh(core_axis_name="c", subcore_axis_name="s",
                                    num_cores=NC, num_subcores=NS),
        scratch_types=[
            pltpu.VMEM((IB,), jnp.int32),
            pltpu.VMEM((IB,), jnp.int32),
            pltpu.VMEM((IB,), jnp.int32),
            pltpu.VMEM((RPB, CH), jnp.int32),
            pltpu.VMEM((RPB, CH), jnp.int32),
        ] + [pltpu.VMEM((CH, M), jnp.float32) for _ in range(PD)]
        + [pltpu.VMEM_SHARED((N_PAD, M), jnp.float32)]
        + [pltpu.SemaphoreType.DMA for _ in range(2 * PD)],
        compiler_params=pltpu.CompilerParams(use_tc_tiling_on_sc=False),
    )


def _gru_body(p_ref, f_ref, wi_ref, wh_ref, wo_ref, bi_ref, bh_ref, bo_ref,
              o_ref):
    m = p_ref[0] + p_ref[1]
    f = f_ref[...]
    gi = jnp.dot(m, wi_ref[...], preferred_element_type=jnp.float32) + bi_ref[...]
    gh = jnp.dot(f, wh_ref[...], preferred_element_type=jnp.float32) + bh_ref[...]
    r = jax.nn.sigmoid(gi[:, :128] + gh[:, :128])
    z = jax.nn.sigmoid(gi[:, 128:256] + gh[:, 128:256])
    n = jnp.tanh(gi[:, 256:] + r * gh[:, 256:])
    h_new = (1.0 - z) * n + z * f
    o_ref[...] = jnp.dot(h_new, wo_ref[...],
                         preferred_element_type=jnp.float32) + bo_ref[...]


def _gru_readout(p2, fp, wi_blk, wh_blk, wo_blk, bi_p, bh_p, bo_p):
    full = lambda j: (0, 0)
    return pl.pallas_call(
        _gru_body,
        grid=(GB,),
        in_specs=[
            pl.BlockSpec((2, BP, 128), lambda j: (0, j, 0)),
            pl.BlockSpec((BP, 128), lambda j: (j, 0)),
            pl.BlockSpec((128, 3 * 128), full),
            pl.BlockSpec((128, 3 * 128), full),
            pl.BlockSpec((128, PK * C), full),
            pl.BlockSpec((1, 3 * 128), full),
            pl.BlockSpec((1, 3 * 128), full),
            pl.BlockSpec((1, PK * C), full),
        ],
        out_specs=pl.BlockSpec((BP, PK * C), lambda j: (j, 0)),
        out_shape=jax.ShapeDtypeStruct((NPK, PK * C), jnp.float32),
    )(p2, fp, wi_blk, wh_blk, wo_blk, bi_p, bh_p, bo_p)


def kernel(features, edge_index, edge_type, edge_table, W_ih, W_hh, b_ih,
           b_hh, W_out, b_out):
    eye8 = jnp.eye(PK, dtype=jnp.float32)
    w2 = edge_table.reshape(T, M, H).transpose(2, 0, 1).reshape(H, T * M)
    w2h = jnp.stack([w2[:, :128], w2[:, 128:]])
    wi_t = W_ih.T
    wh_t = W_hh.T
    wi_blk = jnp.concatenate(
        [jnp.kron(eye8, wi_t[:, g * H:(g + 1) * H]) for g in range(3)], axis=1)
    wh_blk = jnp.concatenate(
        [jnp.kron(eye8, wh_t[:, g * H:(g + 1) * H]) for g in range(3)], axis=1)
    wo_blk = jnp.kron(eye8, W_out.T)
    bi_p = jnp.tile(b_ih.reshape(3, 1, H), (1, PK, 1)).reshape(1, 3 * 128)
    bh_p = jnp.tile(b_hh.reshape(3, 1, H), (1, PK, 1)).reshape(1, 3 * 128)
    bo_p = jnp.tile(b_out.reshape(1, C), (PK, 1)).reshape(1, PK * C)

    pad = E_PAD - E
    padcol = jnp.stack([jnp.zeros((pad,), jnp.int32),
                        jnp.full((pad,), N, jnp.int32)])
    ei1 = jnp.concatenate([edge_index, padcol], axis=1)
    typ1 = jnp.concatenate([edge_type, jnp.zeros((pad,), jnp.int32)])
    zero = jnp.zeros((N_PAD, M), jnp.float32)

    fp = features.reshape(NPK, 128)
    g = _msg_table(features, w2h)
    g1 = g.reshape(N * T, M)

    partials = _sc_gather_scatter()(g1, ei1, typ1, zero)
    p2 = partials.reshape(NC, N_PAD * M // 128, 128)

    out_p = _gru_readout(p2, fp, wi_blk, wh_blk, wo_blk, bi_p, bh_p, bo_p)
    return out_p.reshape(N, C)
